# baseline (speedup 1.0000x reference)
import functools

import jax
import jax.numpy as jnp
from jax import lax
from jax.experimental import pallas as pl
from jax.experimental.pallas import tpu as pltpu
from jax.experimental.pallas import tpu_sc as plsc

N_NODES = 10000
N_PAD = 10112
D = 128
N_EDGES = 320000

NC = 2
NS = 16
NW = NC * NS

CHUNK = 120
NCH0 = 108
NCH1 = 60
TOTAL_CHUNKS = NS * (NCH0 + NCH1)
E_PAD = TOTAL_CHUNKS * CHUNK
ROWS_PER_S = N_PAD // NS

_sc_mesh = plsc.VectorSubcoreMesh(
    core_axis_name="c", subcore_axis_name="s", num_cores=NC, num_subcores=NS
)


@functools.partial(
    pl.kernel,
    out_type=jax.ShapeDtypeStruct((NC * N_PAD, D), jnp.float32),
    mesh=_sc_mesh,
    scratch_types=[
        [pltpu.VMEM((2, CHUNK), jnp.int32) for _ in range(6)],
        [pltpu.VMEM((CHUNK, D), jnp.float32) for _ in range(3)],
        pltpu.VMEM_SHARED((N_PAD, D), jnp.float32),
        pltpu.SemaphoreType.DMA((6,)),
        pltpu.SemaphoreType.DMA((3,)),
        pltpu.SemaphoreType.DMA((3,)),
    ],
)
def _sc_agg(x_hbm, idx_hbm, zeros_hbm, out_hbm, idx_v, rows, agg_sh,
            isem, gsem, ssem):
    c = lax.axis_index("c")
    s = lax.axis_index("s")
    nc = jnp.where(c == 0, NCH0, NCH1)
    cbase = c * NS * NCH0 + s * nc

    pltpu.async_copy(idx_hbm.at[cbase], idx_v[0], isem.at[0])
    pltpu.async_copy(idx_hbm.at[cbase + 1], idx_v[1], isem.at[1])
    pltpu.make_async_copy(idx_hbm.at[cbase], idx_v[0], isem.at[0]).wait()
    pltpu.async_copy(x_hbm.at[idx_v[0].at[0]], rows[0], gsem.at[0])
    row0 = s * ROWS_PER_S
    pltpu.sync_copy(zeros_hbm.at[pl.ds(row0, ROWS_PER_S)],
                    agg_sh.at[pl.ds(row0, ROWS_PER_S)])
    plsc.subcore_barrier()

    def step(o, carry):
        for k in range(6):
            i = o * 6 + k
            b = k % 3
            b1 = (k + 1) % 3
            i6 = k
            n6 = (k + 2) % 6

            @pl.when(i + 2 < nc)
            def _():
                pltpu.async_copy(idx_hbm.at[cbase + i + 2], idx_v[n6],
                                 isem.at[n6])

            @pl.when((i + 1 < nc) & (i >= 2))
            def _():
                pltpu.make_async_copy(rows[b1], agg_sh.at[idx_v[i6].at[1]],
                                      ssem.at[b1]).wait()

            @pl.when(i + 1 < nc)
            def _():
                pltpu.make_async_copy(idx_hbm.at[cbase], idx_v[(k + 1) % 6],
                                      isem.at[(k + 1) % 6]).wait()
                pltpu.async_copy(x_hbm.at[idx_v[(k + 1) % 6].at[0]], rows[b1],
                                 gsem.at[b1])

            pltpu.make_async_copy(x_hbm.at[idx_v[i6].at[0]], rows[b],
                                  gsem.at[b]).wait()
            pltpu.async_copy(rows[b], agg_sh.at[idx_v[i6].at[1]], ssem.at[b],
                             add=True)
        return carry

    lax.fori_loop(0, nc // 6, step, 0)

    for b in range(3):
        pltpu.make_async_copy(rows[b], agg_sh.at[idx_v[0].at[1]],
                              ssem.at[b]).wait()
    plsc.subcore_barrier()

    out_row = c * N_PAD + row0
    pltpu.sync_copy(agg_sh.at[pl.ds(row0, ROWS_PER_S)],
                    out_hbm.at[pl.ds(out_row, ROWS_PER_S)])


BR = 1264


def _dense_body(p_ref, x_ref, wrel_ref, wroot_ref, b_ref, o_ref, *, relu):
    agg = p_ref[0] + p_ref[1]
    acc = jnp.dot(agg, wrel_ref[...], preferred_element_type=jnp.float32)
    acc += jnp.dot(x_ref[...], wroot_ref[...], preferred_element_type=jnp.float32)
    acc += b_ref[...]
    if relu:
        acc = jnp.maximum(acc, 0.0)
    o_ref[...] = acc


def _dense(partials, x, w_rel, w_root, b, relu):
    p3 = partials.reshape(NC, N_PAD, D)
    return pl.pallas_call(
        functools.partial(_dense_body, relu=relu),
        grid=(N_PAD // BR,),
        in_specs=[
            pl.BlockSpec((NC, BR, D), lambda i: (0, i, 0)),
            pl.BlockSpec((BR, D), lambda i: (i, 0)),
            pl.BlockSpec((D, D), lambda i: (0, 0)),
            pl.BlockSpec((D, D), lambda i: (0, 0)),
            pl.BlockSpec((1, D), lambda i: (0, 0)),
        ],
        out_specs=pl.BlockSpec((BR, D), lambda i: (i, 0)),
        out_shape=jax.ShapeDtypeStruct((N_PAD, D), jnp.float32),
    )(p3, x, w_rel, w_root, b.reshape(1, D))


def kernel(x, edge_index, W1_rel, W1_root, b1, W2_rel, W2_root, b2):
    src = edge_index[0].astype(jnp.int32)
    dst = edge_index[1].astype(jnp.int32)
    pad = E_PAD - N_EDGES
    src_p = jnp.concatenate([src, jnp.zeros((pad,), jnp.int32)])
    dst_p = jnp.concatenate([dst, jnp.full((pad,), N_NODES, jnp.int32)])
    src3 = src_p.reshape(TOTAL_CHUNKS, CHUNK)
    dst3 = dst_p.reshape(TOTAL_CHUNKS, CHUNK)
    idx = jnp.stack([src3, dst3], axis=1)
    zeros = jnp.zeros((N_PAD, D), jnp.float32)
    x_pad = jnp.concatenate([x, jnp.zeros((N_PAD - N_NODES, D), jnp.float32)])

    p1 = _sc_agg(x_pad, idx, zeros)
    h = _dense(p1, x_pad, W1_rel, W1_root, b1, relu=True)
    p2 = _sc_agg(h, idx, zeros)
    out = _dense(p2, h, W2_rel, W2_root, b2, relu=False)
    return out[:N_NODES]


# speedup vs baseline: 1.0485x; 1.0485x over previous
import functools

import jax
import jax.numpy as jnp
from jax import lax
from jax.experimental import pallas as pl
from jax.experimental.pallas import tpu as pltpu
from jax.experimental.pallas import tpu_sc as plsc

N_NODES = 10000
N_PAD = 10112
D = 128
N_EDGES = 320000

NC = 2
NS = 16
NW = NC * NS

CHUNK = 120
NCH0 = 120
NCH1 = 48
TOTAL_CHUNKS = NS * (NCH0 + NCH1)
E_PAD = TOTAL_CHUNKS * CHUNK
ROWS_PER_S = N_PAD // NS

_sc_mesh = plsc.VectorSubcoreMesh(
    core_axis_name="c", subcore_axis_name="s", num_cores=NC, num_subcores=NS
)


@functools.partial(
    pl.kernel,
    out_type=jax.ShapeDtypeStruct((NC * N_PAD, D), jnp.float32),
    mesh=_sc_mesh,
    scratch_types=[
        [pltpu.VMEM((2, CHUNK), jnp.int32) for _ in range(6)],
        [pltpu.VMEM((CHUNK, D), jnp.float32) for _ in range(3)],
        pltpu.VMEM_SHARED((N_PAD, D), jnp.float32),
        pltpu.SemaphoreType.DMA((6,)),
        pltpu.SemaphoreType.DMA((3,)),
        pltpu.SemaphoreType.DMA((3,)),
    ],
)
def _sc_agg(x_hbm, idx_hbm, zeros_hbm, out_hbm, idx_v, rows, agg_sh,
            isem, gsem, ssem):
    c = lax.axis_index("c")
    s = lax.axis_index("s")
    nc = jnp.where(c == 0, NCH0, NCH1)
    cbase = c * NS * NCH0 + s * nc

    pltpu.async_copy(idx_hbm.at[cbase], idx_v[0], isem.at[0])
    pltpu.async_copy(idx_hbm.at[cbase + 1], idx_v[1], isem.at[1])
    pltpu.make_async_copy(idx_hbm.at[cbase], idx_v[0], isem.at[0]).wait()
    pltpu.async_copy(x_hbm.at[idx_v[0].at[0]], rows[0], gsem.at[0])
    row0 = s * ROWS_PER_S
    pltpu.sync_copy(zeros_hbm.at[pl.ds(row0, ROWS_PER_S)],
                    agg_sh.at[pl.ds(row0, ROWS_PER_S)])
    plsc.subcore_barrier()

    def step(o, carry):
        for k in range(6):
            i = o * 6 + k
            b = k % 3
            b1 = (k + 1) % 3
            i6 = k
            n6 = (k + 2) % 6

            @pl.when(i + 2 < nc)
            def _():
                pltpu.async_copy(idx_hbm.at[cbase + i + 2], idx_v[n6],
                                 isem.at[n6])

            @pl.when((i + 1 < nc) & (i >= 2))
            def _():
                pltpu.make_async_copy(rows[b1], agg_sh.at[idx_v[i6].at[1]],
                                      ssem.at[b1]).wait()

            @pl.when(i + 1 < nc)
            def _():
                pltpu.make_async_copy(idx_hbm.at[cbase], idx_v[(k + 1) % 6],
                                      isem.at[(k + 1) % 6]).wait()
                pltpu.async_copy(x_hbm.at[idx_v[(k + 1) % 6].at[0]], rows[b1],
                                 gsem.at[b1])

            pltpu.make_async_copy(x_hbm.at[idx_v[i6].at[0]], rows[b],
                                  gsem.at[b]).wait()
            pltpu.async_copy(rows[b], agg_sh.at[idx_v[i6].at[1]], ssem.at[b],
                             add=True)
        return carry

    lax.fori_loop(0, nc // 6, step, 0)

    for b in range(3):
        pltpu.make_async_copy(rows[b], agg_sh.at[idx_v[0].at[1]],
                              ssem.at[b]).wait()
    plsc.subcore_barrier()

    out_row = c * N_PAD + row0
    pltpu.sync_copy(agg_sh.at[pl.ds(row0, ROWS_PER_S)],
                    out_hbm.at[pl.ds(out_row, ROWS_PER_S)])


BR = 1264


def _dense_body(p_ref, x_ref, wrel_ref, wroot_ref, b_ref, o_ref, *, relu):
    agg = p_ref[0] + p_ref[1]
    acc = jnp.dot(agg, wrel_ref[...], preferred_element_type=jnp.float32)
    acc += jnp.dot(x_ref[...], wroot_ref[...], preferred_element_type=jnp.float32)
    acc += b_ref[...]
    if relu:
        acc = jnp.maximum(acc, 0.0)
    o_ref[...] = acc


def _dense(partials, x, w_rel, w_root, b, relu):
    p3 = partials.reshape(NC, N_PAD, D)
    return pl.pallas_call(
        functools.partial(_dense_body, relu=relu),
        grid=(N_PAD // BR,),
        in_specs=[
            pl.BlockSpec((NC, BR, D), lambda i: (0, i, 0)),
            pl.BlockSpec((BR, D), lambda i: (i, 0)),
            pl.BlockSpec((D, D), lambda i: (0, 0)),
            pl.BlockSpec((D, D), lambda i: (0, 0)),
            pl.BlockSpec((1, D), lambda i: (0, 0)),
        ],
        out_specs=pl.BlockSpec((BR, D), lambda i: (i, 0)),
        out_shape=jax.ShapeDtypeStruct((N_PAD, D), jnp.float32),
    )(p3, x, w_rel, w_root, b.reshape(1, D))


def kernel(x, edge_index, W1_rel, W1_root, b1, W2_rel, W2_root, b2):
    src = edge_index[0].astype(jnp.int32)
    dst = edge_index[1].astype(jnp.int32)
    pad = E_PAD - N_EDGES
    src_p = jnp.concatenate([src, jnp.zeros((pad,), jnp.int32)])
    dst_p = jnp.concatenate([dst, jnp.full((pad,), N_NODES, jnp.int32)])
    src3 = src_p.reshape(TOTAL_CHUNKS, CHUNK)
    dst3 = dst_p.reshape(TOTAL_CHUNKS, CHUNK)
    idx = jnp.stack([src3, dst3], axis=1)
    zeros = jnp.zeros((N_PAD, D), jnp.float32)
    x_pad = jnp.concatenate([x, jnp.zeros((N_PAD - N_NODES, D), jnp.float32)])

    p1 = _sc_agg(x_pad, idx, zeros)
    h = _dense(p1, x_pad, W1_rel, W1_root, b1, relu=True)
    p2 = _sc_agg(h, idx, zeros)
    out = _dense(p2, h, W2_rel, W2_root, b2, relu=False)
    return out[:N_NODES]


# speedup vs baseline: 1.0682x; 1.0188x over previous
import functools

import jax
import jax.numpy as jnp
from jax import lax
from jax.experimental import pallas as pl
from jax.experimental.pallas import tpu as pltpu
from jax.experimental.pallas import tpu_sc as plsc

N_NODES = 10000
N_PAD = 10112
D = 128
N_EDGES = 320000

NC = 2
NS = 16
NW = NC * NS

CHUNK = 120
NCH0 = 126
NCH1 = 42
TOTAL_CHUNKS = NS * (NCH0 + NCH1)
E_PAD = TOTAL_CHUNKS * CHUNK
ROWS_PER_S = N_PAD // NS

_sc_mesh = plsc.VectorSubcoreMesh(
    core_axis_name="c", subcore_axis_name="s", num_cores=NC, num_subcores=NS
)


@functools.partial(
    pl.kernel,
    out_type=jax.ShapeDtypeStruct((NC * N_PAD, D), jnp.float32),
    mesh=_sc_mesh,
    scratch_types=[
        [pltpu.VMEM((2, CHUNK), jnp.int32) for _ in range(6)],
        [pltpu.VMEM((CHUNK, D), jnp.float32) for _ in range(3)],
        pltpu.VMEM_SHARED((N_PAD, D), jnp.float32),
        pltpu.SemaphoreType.DMA((6,)),
        pltpu.SemaphoreType.DMA((3,)),
        pltpu.SemaphoreType.DMA((3,)),
    ],
)
def _sc_agg(x_hbm, idx_hbm, zeros_hbm, out_hbm, idx_v, rows, agg_sh,
            isem, gsem, ssem):
    c = lax.axis_index("c")
    s = lax.axis_index("s")
    nc = jnp.where(c == 0, NCH0, NCH1)
    cbase = c * NS * NCH0 + s * nc

    pltpu.async_copy(idx_hbm.at[cbase], idx_v[0], isem.at[0])
    pltpu.async_copy(idx_hbm.at[cbase + 1], idx_v[1], isem.at[1])
    pltpu.make_async_copy(idx_hbm.at[cbase], idx_v[0], isem.at[0]).wait()
    pltpu.async_copy(x_hbm.at[idx_v[0].at[0]], rows[0], gsem.at[0])
    row0 = s * ROWS_PER_S
    pltpu.sync_copy(zeros_hbm.at[pl.ds(row0, ROWS_PER_S)],
                    agg_sh.at[pl.ds(row0, ROWS_PER_S)])
    plsc.subcore_barrier()

    def step(o, carry):
        for k in range(6):
            i = o * 6 + k
            b = k % 3
            b1 = (k + 1) % 3
            i6 = k
            n6 = (k + 2) % 6

            @pl.when(i + 2 < nc)
            def _():
                pltpu.async_copy(idx_hbm.at[cbase + i + 2], idx_v[n6],
                                 isem.at[n6])

            @pl.when((i + 1 < nc) & (i >= 2))
            def _():
                pltpu.make_async_copy(rows[b1], agg_sh.at[idx_v[i6].at[1]],
                                      ssem.at[b1]).wait()

            @pl.when(i + 1 < nc)
            def _():
                pltpu.make_async_copy(idx_hbm.at[cbase], idx_v[(k + 1) % 6],
                                      isem.at[(k + 1) % 6]).wait()
                pltpu.async_copy(x_hbm.at[idx_v[(k + 1) % 6].at[0]], rows[b1],
                                 gsem.at[b1])

            pltpu.make_async_copy(x_hbm.at[idx_v[i6].at[0]], rows[b],
                                  gsem.at[b]).wait()
            pltpu.async_copy(rows[b], agg_sh.at[idx_v[i6].at[1]], ssem.at[b],
                             add=True)
        return carry

    lax.fori_loop(0, nc // 6, step, 0)

    for b in range(3):
        pltpu.make_async_copy(rows[b], agg_sh.at[idx_v[0].at[1]],
                              ssem.at[b]).wait()
    plsc.subcore_barrier()

    out_row = c * N_PAD + row0
    pltpu.sync_copy(agg_sh.at[pl.ds(row0, ROWS_PER_S)],
                    out_hbm.at[pl.ds(out_row, ROWS_PER_S)])


BR = 1264


def _dense_body(p_ref, x_ref, wrel_ref, wroot_ref, b_ref, o_ref, *, relu):
    agg = p_ref[0] + p_ref[1]
    acc = jnp.dot(agg, wrel_ref[...], preferred_element_type=jnp.float32)
    acc += jnp.dot(x_ref[...], wroot_ref[...], preferred_element_type=jnp.float32)
    acc += b_ref[...]
    if relu:
        acc = jnp.maximum(acc, 0.0)
    o_ref[...] = acc


def _dense(partials, x, w_rel, w_root, b, relu):
    p3 = partials.reshape(NC, N_PAD, D)
    return pl.pallas_call(
        functools.partial(_dense_body, relu=relu),
        grid=(N_PAD // BR,),
        in_specs=[
            pl.BlockSpec((NC, BR, D), lambda i: (0, i, 0)),
            pl.BlockSpec((BR, D), lambda i: (i, 0)),
            pl.BlockSpec((D, D), lambda i: (0, 0)),
            pl.BlockSpec((D, D), lambda i: (0, 0)),
            pl.BlockSpec((1, D), lambda i: (0, 0)),
        ],
        out_specs=pl.BlockSpec((BR, D), lambda i: (i, 0)),
        out_shape=jax.ShapeDtypeStruct((N_PAD, D), jnp.float32),
    )(p3, x, w_rel, w_root, b.reshape(1, D))


def kernel(x, edge_index, W1_rel, W1_root, b1, W2_rel, W2_root, b2):
    src = edge_index[0].astype(jnp.int32)
    dst = edge_index[1].astype(jnp.int32)
    pad = E_PAD - N_EDGES
    src_p = jnp.concatenate([src, jnp.zeros((pad,), jnp.int32)])
    dst_p = jnp.concatenate([dst, jnp.full((pad,), N_NODES, jnp.int32)])
    src3 = src_p.reshape(TOTAL_CHUNKS, CHUNK)
    dst3 = dst_p.reshape(TOTAL_CHUNKS, CHUNK)
    idx = jnp.stack([src3, dst3], axis=1)
    zeros = jnp.zeros((N_PAD, D), jnp.float32)
    x_pad = jnp.concatenate([x, jnp.zeros((N_PAD - N_NODES, D), jnp.float32)])

    p1 = _sc_agg(x_pad, idx, zeros)
    h = _dense(p1, x_pad, W1_rel, W1_root, b1, relu=True)
    p2 = _sc_agg(h, idx, zeros)
    out = _dense(p2, h, W2_rel, W2_root, b2, relu=False)
    return out[:N_NODES]


# speedup vs baseline: 1.0961x; 1.0261x over previous
import functools

import jax
import jax.numpy as jnp
from jax import lax
from jax.experimental import pallas as pl
from jax.experimental.pallas import tpu as pltpu
from jax.experimental.pallas import tpu_sc as plsc

N_NODES = 10000
N_PAD = 10112
D = 128
N_EDGES = 320000

NC = 2
NS = 16
NW = NC * NS

CHUNK = 120
NCH0 = 132
NCH1 = 36
TOTAL_CHUNKS = NS * (NCH0 + NCH1)
E_PAD = TOTAL_CHUNKS * CHUNK
ROWS_PER_S = N_PAD // NS

_sc_mesh = plsc.VectorSubcoreMesh(
    core_axis_name="c", subcore_axis_name="s", num_cores=NC, num_subcores=NS
)


@functools.partial(
    pl.kernel,
    out_type=jax.ShapeDtypeStruct((NC * N_PAD, D), jnp.float32),
    mesh=_sc_mesh,
    scratch_types=[
        [pltpu.VMEM((2, CHUNK), jnp.int32) for _ in range(6)],
        [pltpu.VMEM((CHUNK, D), jnp.float32) for _ in range(3)],
        pltpu.VMEM_SHARED((N_PAD, D), jnp.float32),
        pltpu.SemaphoreType.DMA((6,)),
        pltpu.SemaphoreType.DMA((3,)),
        pltpu.SemaphoreType.DMA((3,)),
    ],
)
def _sc_agg(x_hbm, idx_hbm, zeros_hbm, out_hbm, idx_v, rows, agg_sh,
            isem, gsem, ssem):
    c = lax.axis_index("c")
    s = lax.axis_index("s")
    nc = jnp.where(c == 0, NCH0, NCH1)
    cbase = c * NS * NCH0 + s * nc

    pltpu.async_copy(idx_hbm.at[cbase], idx_v[0], isem.at[0])
    pltpu.async_copy(idx_hbm.at[cbase + 1], idx_v[1], isem.at[1])
    pltpu.make_async_copy(idx_hbm.at[cbase], idx_v[0], isem.at[0]).wait()
    pltpu.async_copy(x_hbm.at[idx_v[0].at[0]], rows[0], gsem.at[0])
    row0 = s * ROWS_PER_S
    pltpu.sync_copy(zeros_hbm.at[pl.ds(row0, ROWS_PER_S)],
                    agg_sh.at[pl.ds(row0, ROWS_PER_S)])
    plsc.subcore_barrier()

    def step(o, carry):
        for k in range(6):
            i = o * 6 + k
            b = k % 3
            b1 = (k + 1) % 3
            i6 = k
            n6 = (k + 2) % 6

            @pl.when(i + 2 < nc)
            def _():
                pltpu.async_copy(idx_hbm.at[cbase + i + 2], idx_v[n6],
                                 isem.at[n6])

            @pl.when((i + 1 < nc) & (i >= 2))
            def _():
                pltpu.make_async_copy(rows[b1], agg_sh.at[idx_v[i6].at[1]],
                                      ssem.at[b1]).wait()

            @pl.when(i + 1 < nc)
            def _():
                pltpu.make_async_copy(idx_hbm.at[cbase], idx_v[(k + 1) % 6],
                                      isem.at[(k + 1) % 6]).wait()
                pltpu.async_copy(x_hbm.at[idx_v[(k + 1) % 6].at[0]], rows[b1],
                                 gsem.at[b1])

            pltpu.make_async_copy(x_hbm.at[idx_v[i6].at[0]], rows[b],
                                  gsem.at[b]).wait()
            pltpu.async_copy(rows[b], agg_sh.at[idx_v[i6].at[1]], ssem.at[b],
                             add=True)
        return carry

    lax.fori_loop(0, nc // 6, step, 0)

    for b in range(3):
        pltpu.make_async_copy(rows[b], agg_sh.at[idx_v[0].at[1]],
                              ssem.at[b]).wait()
    plsc.subcore_barrier()

    out_row = c * N_PAD + row0
    pltpu.sync_copy(agg_sh.at[pl.ds(row0, ROWS_PER_S)],
                    out_hbm.at[pl.ds(out_row, ROWS_PER_S)])


BR = 1264


def _dense_body(p_ref, x_ref, wrel_ref, wroot_ref, b_ref, o_ref, *, relu):
    agg = p_ref[0] + p_ref[1]
    acc = jnp.dot(agg, wrel_ref[...], preferred_element_type=jnp.float32)
    acc += jnp.dot(x_ref[...], wroot_ref[...], preferred_element_type=jnp.float32)
    acc += b_ref[...]
    if relu:
        acc = jnp.maximum(acc, 0.0)
    o_ref[...] = acc


def _dense(partials, x, w_rel, w_root, b, relu):
    p3 = partials.reshape(NC, N_PAD, D)
    return pl.pallas_call(
        functools.partial(_dense_body, relu=relu),
        grid=(N_PAD // BR,),
        in_specs=[
            pl.BlockSpec((NC, BR, D), lambda i: (0, i, 0)),
            pl.BlockSpec((BR, D), lambda i: (i, 0)),
            pl.BlockSpec((D, D), lambda i: (0, 0)),
            pl.BlockSpec((D, D), lambda i: (0, 0)),
            pl.BlockSpec((1, D), lambda i: (0, 0)),
        ],
        out_specs=pl.BlockSpec((BR, D), lambda i: (i, 0)),
        out_shape=jax.ShapeDtypeStruct((N_PAD, D), jnp.float32),
    )(p3, x, w_rel, w_root, b.reshape(1, D))


def kernel(x, edge_index, W1_rel, W1_root, b1, W2_rel, W2_root, b2):
    src = edge_index[0].astype(jnp.int32)
    dst = edge_index[1].astype(jnp.int32)
    pad = E_PAD - N_EDGES
    src_p = jnp.concatenate([src, jnp.zeros((pad,), jnp.int32)])
    dst_p = jnp.concatenate([dst, jnp.full((pad,), N_NODES, jnp.int32)])
    src3 = src_p.reshape(TOTAL_CHUNKS, CHUNK)
    dst3 = dst_p.reshape(TOTAL_CHUNKS, CHUNK)
    idx = jnp.stack([src3, dst3], axis=1)
    zeros = jnp.zeros((N_PAD, D), jnp.float32)
    x_pad = jnp.concatenate([x, jnp.zeros((N_PAD - N_NODES, D), jnp.float32)])

    p1 = _sc_agg(x_pad, idx, zeros)
    h = _dense(p1, x_pad, W1_rel, W1_root, b1, relu=True)
    p2 = _sc_agg(h, idx, zeros)
    out = _dense(p2, h, W2_rel, W2_root, b2, relu=False)
    return out[:N_NODES]


# speedup vs baseline: 1.1408x; 1.0408x over previous
import functools

import jax
import jax.numpy as jnp
from jax import lax
from jax.experimental import pallas as pl
from jax.experimental.pallas import tpu as pltpu
from jax.experimental.pallas import tpu_sc as plsc

N_NODES = 10000
N_PAD = 10112
D = 128
N_EDGES = 320000

NC = 2
NS = 16
NW = NC * NS

CHUNK = 120
NCH0 = 144
NCH1 = 24
TOTAL_CHUNKS = NS * (NCH0 + NCH1)
E_PAD = TOTAL_CHUNKS * CHUNK
ROWS_PER_S = N_PAD // NS

_sc_mesh = plsc.VectorSubcoreMesh(
    core_axis_name="c", subcore_axis_name="s", num_cores=NC, num_subcores=NS
)


@functools.partial(
    pl.kernel,
    out_type=jax.ShapeDtypeStruct((NC * N_PAD, D), jnp.float32),
    mesh=_sc_mesh,
    scratch_types=[
        [pltpu.VMEM((2, CHUNK), jnp.int32) for _ in range(6)],
        [pltpu.VMEM((CHUNK, D), jnp.float32) for _ in range(3)],
        pltpu.VMEM_SHARED((N_PAD, D), jnp.float32),
        pltpu.SemaphoreType.DMA((6,)),
        pltpu.SemaphoreType.DMA((3,)),
        pltpu.SemaphoreType.DMA((3,)),
    ],
)
def _sc_agg(x_hbm, idx_hbm, zeros_hbm, out_hbm, idx_v, rows, agg_sh,
            isem, gsem, ssem):
    c = lax.axis_index("c")
    s = lax.axis_index("s")
    nc = jnp.where(c == 0, NCH0, NCH1)
    cbase = c * NS * NCH0 + s * nc

    pltpu.async_copy(idx_hbm.at[cbase], idx_v[0], isem.at[0])
    pltpu.async_copy(idx_hbm.at[cbase + 1], idx_v[1], isem.at[1])
    pltpu.make_async_copy(idx_hbm.at[cbase], idx_v[0], isem.at[0]).wait()
    pltpu.async_copy(x_hbm.at[idx_v[0].at[0]], rows[0], gsem.at[0])
    row0 = s * ROWS_PER_S
    pltpu.sync_copy(zeros_hbm.at[pl.ds(row0, ROWS_PER_S)],
                    agg_sh.at[pl.ds(row0, ROWS_PER_S)])
    plsc.subcore_barrier()

    def step(o, carry):
        for k in range(6):
            i = o * 6 + k
            b = k % 3
            b1 = (k + 1) % 3
            i6 = k
            n6 = (k + 2) % 6

            @pl.when(i + 2 < nc)
            def _():
                pltpu.async_copy(idx_hbm.at[cbase + i + 2], idx_v[n6],
                                 isem.at[n6])

            @pl.when((i + 1 < nc) & (i >= 2))
            def _():
                pltpu.make_async_copy(rows[b1], agg_sh.at[idx_v[i6].at[1]],
                                      ssem.at[b1]).wait()

            @pl.when(i + 1 < nc)
            def _():
                pltpu.make_async_copy(idx_hbm.at[cbase], idx_v[(k + 1) % 6],
                                      isem.at[(k + 1) % 6]).wait()
                pltpu.async_copy(x_hbm.at[idx_v[(k + 1) % 6].at[0]], rows[b1],
                                 gsem.at[b1])

            pltpu.make_async_copy(x_hbm.at[idx_v[i6].at[0]], rows[b],
                                  gsem.at[b]).wait()
            pltpu.async_copy(rows[b], agg_sh.at[idx_v[i6].at[1]], ssem.at[b],
                             add=True)
        return carry

    lax.fori_loop(0, nc // 6, step, 0)

    for b in range(3):
        pltpu.make_async_copy(rows[b], agg_sh.at[idx_v[0].at[1]],
                              ssem.at[b]).wait()
    plsc.subcore_barrier()

    out_row = c * N_PAD + row0
    pltpu.sync_copy(agg_sh.at[pl.ds(row0, ROWS_PER_S)],
                    out_hbm.at[pl.ds(out_row, ROWS_PER_S)])


BR = 1264


def _dense_body(p_ref, x_ref, wrel_ref, wroot_ref, b_ref, o_ref, *, relu):
    agg = p_ref[0] + p_ref[1]
    acc = jnp.dot(agg, wrel_ref[...], preferred_element_type=jnp.float32)
    acc += jnp.dot(x_ref[...], wroot_ref[...], preferred_element_type=jnp.float32)
    acc += b_ref[...]
    if relu:
        acc = jnp.maximum(acc, 0.0)
    o_ref[...] = acc


def _dense(partials, x, w_rel, w_root, b, relu):
    p3 = partials.reshape(NC, N_PAD, D)
    return pl.pallas_call(
        functools.partial(_dense_body, relu=relu),
        grid=(N_PAD // BR,),
        in_specs=[
            pl.BlockSpec((NC, BR, D), lambda i: (0, i, 0)),
            pl.BlockSpec((BR, D), lambda i: (i, 0)),
            pl.BlockSpec((D, D), lambda i: (0, 0)),
            pl.BlockSpec((D, D), lambda i: (0, 0)),
            pl.BlockSpec((1, D), lambda i: (0, 0)),
        ],
        out_specs=pl.BlockSpec((BR, D), lambda i: (i, 0)),
        out_shape=jax.ShapeDtypeStruct((N_PAD, D), jnp.float32),
    )(p3, x, w_rel, w_root, b.reshape(1, D))


def kernel(x, edge_index, W1_rel, W1_root, b1, W2_rel, W2_root, b2):
    src = edge_index[0].astype(jnp.int32)
    dst = edge_index[1].astype(jnp.int32)
    pad = E_PAD - N_EDGES
    src_p = jnp.concatenate([src, jnp.zeros((pad,), jnp.int32)])
    dst_p = jnp.concatenate([dst, jnp.full((pad,), N_NODES, jnp.int32)])
    src3 = src_p.reshape(TOTAL_CHUNKS, CHUNK)
    dst3 = dst_p.reshape(TOTAL_CHUNKS, CHUNK)
    idx = jnp.stack([src3, dst3], axis=1)
    zeros = jnp.zeros((N_PAD, D), jnp.float32)
    x_pad = jnp.concatenate([x, jnp.zeros((N_PAD - N_NODES, D), jnp.float32)])

    p1 = _sc_agg(x_pad, idx, zeros)
    h = _dense(p1, x_pad, W1_rel, W1_root, b1, relu=True)
    p2 = _sc_agg(h, idx, zeros)
    out = _dense(p2, h, W2_rel, W2_root, b2, relu=False)
    return out[:N_NODES]


# speedup vs baseline: 1.1614x; 1.0181x over previous
import functools

import jax
import jax.numpy as jnp
from jax import lax
from jax.experimental import pallas as pl
from jax.experimental.pallas import tpu as pltpu
from jax.experimental.pallas import tpu_sc as plsc

N_NODES = 10000
N_PAD = 10112
D = 128
N_EDGES = 320000

NC = 2
NS = 16
NW = NC * NS

CHUNK = 120
NCH0 = 156
NCH1 = 12
TOTAL_CHUNKS = NS * (NCH0 + NCH1)
E_PAD = TOTAL_CHUNKS * CHUNK
ROWS_PER_S = N_PAD // NS

_sc_mesh = plsc.VectorSubcoreMesh(
    core_axis_name="c", subcore_axis_name="s", num_cores=NC, num_subcores=NS
)


@functools.partial(
    pl.kernel,
    out_type=jax.ShapeDtypeStruct((NC * N_PAD, D), jnp.float32),
    mesh=_sc_mesh,
    scratch_types=[
        [pltpu.VMEM((2, CHUNK), jnp.int32) for _ in range(6)],
        [pltpu.VMEM((CHUNK, D), jnp.float32) for _ in range(3)],
        pltpu.VMEM_SHARED((N_PAD, D), jnp.float32),
        pltpu.SemaphoreType.DMA((6,)),
        pltpu.SemaphoreType.DMA((3,)),
        pltpu.SemaphoreType.DMA((3,)),
    ],
)
def _sc_agg(x_hbm, idx_hbm, zeros_hbm, out_hbm, idx_v, rows, agg_sh,
            isem, gsem, ssem):
    c = lax.axis_index("c")
    s = lax.axis_index("s")
    nc = jnp.where(c == 0, NCH0, NCH1)
    cbase = c * NS * NCH0 + s * nc

    pltpu.async_copy(idx_hbm.at[cbase], idx_v[0], isem.at[0])
    pltpu.async_copy(idx_hbm.at[cbase + 1], idx_v[1], isem.at[1])
    pltpu.make_async_copy(idx_hbm.at[cbase], idx_v[0], isem.at[0]).wait()
    pltpu.async_copy(x_hbm.at[idx_v[0].at[0]], rows[0], gsem.at[0])
    row0 = s * ROWS_PER_S
    pltpu.sync_copy(zeros_hbm.at[pl.ds(row0, ROWS_PER_S)],
                    agg_sh.at[pl.ds(row0, ROWS_PER_S)])
    plsc.subcore_barrier()

    def step(o, carry):
        for k in range(6):
            i = o * 6 + k
            b = k % 3
            b1 = (k + 1) % 3
            i6 = k
            n6 = (k + 2) % 6

            @pl.when(i + 2 < nc)
            def _():
                pltpu.async_copy(idx_hbm.at[cbase + i + 2], idx_v[n6],
                                 isem.at[n6])

            @pl.when((i + 1 < nc) & (i >= 2))
            def _():
                pltpu.make_async_copy(rows[b1], agg_sh.at[idx_v[i6].at[1]],
                                      ssem.at[b1]).wait()

            @pl.when(i + 1 < nc)
            def _():
                pltpu.make_async_copy(idx_hbm.at[cbase], idx_v[(k + 1) % 6],
                                      isem.at[(k + 1) % 6]).wait()
                pltpu.async_copy(x_hbm.at[idx_v[(k + 1) % 6].at[0]], rows[b1],
                                 gsem.at[b1])

            pltpu.make_async_copy(x_hbm.at[idx_v[i6].at[0]], rows[b],
                                  gsem.at[b]).wait()
            pltpu.async_copy(rows[b], agg_sh.at[idx_v[i6].at[1]], ssem.at[b],
                             add=True)
        return carry

    lax.fori_loop(0, nc // 6, step, 0)

    for b in range(3):
        pltpu.make_async_copy(rows[b], agg_sh.at[idx_v[0].at[1]],
                              ssem.at[b]).wait()
    plsc.subcore_barrier()

    out_row = c * N_PAD + row0
    pltpu.sync_copy(agg_sh.at[pl.ds(row0, ROWS_PER_S)],
                    out_hbm.at[pl.ds(out_row, ROWS_PER_S)])


BR = 1264


def _dense_body(p_ref, x_ref, wrel_ref, wroot_ref, b_ref, o_ref, *, relu):
    agg = p_ref[0] + p_ref[1]
    acc = jnp.dot(agg, wrel_ref[...], preferred_element_type=jnp.float32)
    acc += jnp.dot(x_ref[...], wroot_ref[...], preferred_element_type=jnp.float32)
    acc += b_ref[...]
    if relu:
        acc = jnp.maximum(acc, 0.0)
    o_ref[...] = acc


def _dense(partials, x, w_rel, w_root, b, relu):
    p3 = partials.reshape(NC, N_PAD, D)
    return pl.pallas_call(
        functools.partial(_dense_body, relu=relu),
        grid=(N_PAD // BR,),
        in_specs=[
            pl.BlockSpec((NC, BR, D), lambda i: (0, i, 0)),
            pl.BlockSpec((BR, D), lambda i: (i, 0)),
            pl.BlockSpec((D, D), lambda i: (0, 0)),
            pl.BlockSpec((D, D), lambda i: (0, 0)),
            pl.BlockSpec((1, D), lambda i: (0, 0)),
        ],
        out_specs=pl.BlockSpec((BR, D), lambda i: (i, 0)),
        out_shape=jax.ShapeDtypeStruct((N_PAD, D), jnp.float32),
    )(p3, x, w_rel, w_root, b.reshape(1, D))


def kernel(x, edge_index, W1_rel, W1_root, b1, W2_rel, W2_root, b2):
    src = edge_index[0].astype(jnp.int32)
    dst = edge_index[1].astype(jnp.int32)
    pad = E_PAD - N_EDGES
    src_p = jnp.concatenate([src, jnp.zeros((pad,), jnp.int32)])
    dst_p = jnp.concatenate([dst, jnp.full((pad,), N_NODES, jnp.int32)])
    src3 = src_p.reshape(TOTAL_CHUNKS, CHUNK)
    dst3 = dst_p.reshape(TOTAL_CHUNKS, CHUNK)
    idx = jnp.stack([src3, dst3], axis=1)
    zeros = jnp.zeros((N_PAD, D), jnp.float32)
    x_pad = jnp.concatenate([x, jnp.zeros((N_PAD - N_NODES, D), jnp.float32)])

    p1 = _sc_agg(x_pad, idx, zeros)
    h = _dense(p1, x_pad, W1_rel, W1_root, b1, relu=True)
    p2 = _sc_agg(h, idx, zeros)
    out = _dense(p2, h, W2_rel, W2_root, b2, relu=False)
    return out[:N_NODES]
